# baseline (speedup 1.0000x reference)
import functools

import jax
import jax.numpy as jnp
from jax import lax
from jax.experimental import pallas as pl
from jax.experimental.pallas import tpu as pltpu
from jax.experimental.pallas import tpu_sc as plsc

T = 4096
DIM = 2048
HID = 1024
E = 16
K = 2
TK = T * K
BM = 128
NB = TK // BM
W = NB + E

_F32 = jnp.float32
_BF16 = jnp.bfloat16
_I32 = jnp.int32


def _mm_t(a, b):
    return lax.dot_general(a, b, (((1,), (1,)), ((), ())),
                           preferred_element_type=_F32)


def _router_body(x_ref, gw_ref, bias_ref, e1_ref, e2_ref, s1_ref, s2_ref):
    x = x_ref[...]
    logits = _mm_t(x, gw_ref[...])
    scores = jax.nn.sigmoid(logits)
    biased = scores + bias_ref[0:1, :]
    iota_e = lax.broadcasted_iota(_I32, (1, E), 1)
    m1 = jnp.max(biased, axis=1, keepdims=True)
    a1 = jnp.min(jnp.where(biased == m1, iota_e, E), axis=1, keepdims=True)
    masked = jnp.where(iota_e == a1, -jnp.inf, biased)
    m2 = jnp.max(masked, axis=1, keepdims=True)
    a2 = jnp.min(jnp.where(masked == m2, iota_e, E), axis=1, keepdims=True)
    s1 = jnp.sum(jnp.where(iota_e == a1, scores, 0.0), axis=1, keepdims=True)
    s2 = jnp.sum(jnp.where(iota_e == a2, scores, 0.0), axis=1, keepdims=True)
    den = s1 + s2 + 1e-20
    e1_ref[...] = a1
    e2_ref[...] = a2
    s1_ref[...] = s1 / den
    s2_ref[...] = s2 / den


def _router(x, gate_w, bias8):
    bm = 1024
    grid = (T // bm,)
    out_shape = (
        jax.ShapeDtypeStruct((T, 1), _I32),
        jax.ShapeDtypeStruct((T, 1), _I32),
        jax.ShapeDtypeStruct((T, 1), _F32),
        jax.ShapeDtypeStruct((T, 1), _F32),
    )
    row_spec = pl.BlockSpec((bm, 1), lambda i: (i, 0))
    return pl.pallas_call(
        _router_body,
        grid=grid,
        in_specs=[
            pl.BlockSpec((bm, DIM), lambda i: (i, 0)),
            pl.BlockSpec((E, DIM), lambda i: (0, 0)),
            pl.BlockSpec((8, E), lambda i: (0, 0)),
        ],
        out_specs=(row_spec, row_spec, row_spec, row_spec),
        out_shape=out_shape,
    )(x, gate_w, bias8)


def _sort_body(e1_ref, e2_ref, d1_ref, d2_ref, offs_ref, grp_ref, blk_ref):
    iota_e = lax.broadcasted_iota(_I32, (1, E), 1)
    oh1 = (e1_ref[...] == iota_e).astype(_I32)
    oh2 = (e2_ref[...] == iota_e).astype(_I32)
    c = oh1 + oh2
    s = 1
    while s < T:
        c = c + jnp.concatenate(
            [jnp.zeros((s, E), _I32), c[: T - s]], axis=0)
        s *= 2
    total = c[T - 1: T, :]
    cnt_before = c - oh1 - oh2
    oi = total
    s = 1
    while s < E:
        oi = oi + jnp.concatenate(
            [jnp.zeros((1, s), _I32), oi[:, : E - s]], axis=1)
        s *= 2
    off_excl = oi - total
    d1_ref[...] = jnp.sum(oh1 * (off_excl + cnt_before), axis=1, keepdims=True)
    d2_ref[...] = jnp.sum(oh2 * (off_excl + cnt_before), axis=1, keepdims=True)
    offs = jnp.concatenate(
        [off_excl, jnp.full((1, 2), TK, _I32)], axis=1)
    offs_ref[...] = jnp.broadcast_to(offs, (8, E + 2))
    nz = total > 0
    fb = off_excl // BM
    lb = (jnp.maximum(oi, 1) - 1) // BM
    tiles = jnp.where(nz, lb - fb + 1, 0)
    cti = tiles
    s = 1
    while s < E:
        cti = cti + jnp.concatenate(
            [jnp.zeros((1, s), _I32), cti[:, : E - s]], axis=1)
        s *= 2
    cte = cti - tiles
    item = lax.broadcasted_iota(_I32, (W, 1), 0)
    gof = jnp.sum((cti <= item).astype(_I32), axis=1, keepdims=True)
    ohg = lax.broadcasted_iota(_I32, (W, E), 1) == gof
    blk = jnp.sum(jnp.where(ohg, fb - cte, 0), axis=1, keepdims=True) + item
    blk_ref[...] = jnp.where(gof >= E, NB - 1, blk)
    grp_ref[...] = jnp.minimum(gof, E)


def _sort(e1, e2):
    out_shape = (
        jax.ShapeDtypeStruct((T, 1), _I32),
        jax.ShapeDtypeStruct((T, 1), _I32),
        jax.ShapeDtypeStruct((8, E + 2), _I32),
        jax.ShapeDtypeStruct((W, 1), _I32),
        jax.ShapeDtypeStruct((W, 1), _I32),
    )
    return pl.pallas_call(_sort_body, out_shape=out_shape)(e1, e2)


_SC_NW = 32
_SC_CH = 32


def _sc_scatter(x, d1, d2):
    mesh = plsc.VectorSubcoreMesh(core_axis_name="c", subcore_axis_name="s")
    per_w = T // _SC_NW

    @functools.partial(
        pl.kernel,
        out_type=jax.ShapeDtypeStruct((TK, DIM), _F32),
        mesh=mesh,
        scratch_types=[
            pltpu.VMEM((_SC_CH,), _I32),
            pltpu.VMEM((_SC_CH,), _I32),
            pltpu.VMEM((_SC_CH, DIM), _F32),
            pltpu.SemaphoreType.DMA,
            pltpu.SemaphoreType.DMA,
        ],
    )
    def scatter_k(x_hbm, d1_hbm, d2_hbm, out_hbm, i1_v, i2_v, rows_v,
                  sem1, sem2):
        wid = lax.axis_index("s") * 2 + lax.axis_index("c")
        base = wid * per_w

        def body(j, carry):
            b = base + j * _SC_CH
            pltpu.sync_copy(x_hbm.at[pl.ds(b, _SC_CH)], rows_v)
            pltpu.sync_copy(d1_hbm.at[pl.ds(b, _SC_CH)], i1_v)
            pltpu.sync_copy(d2_hbm.at[pl.ds(b, _SC_CH)], i2_v)
            c1 = pltpu.async_copy(rows_v, out_hbm.at[i1_v], sem1)
            c2 = pltpu.async_copy(rows_v, out_hbm.at[i2_v], sem2)
            c1.wait()
            c2.wait()
            return carry

        lax.fori_loop(0, per_w // _SC_CH, body, 0)

    return scatter_k(x, d1, d2)


def _sc_gather(eo, d1, d2):
    mesh = plsc.VectorSubcoreMesh(core_axis_name="c", subcore_axis_name="s")
    per_w = T // _SC_NW
    row_t = jax.ShapeDtypeStruct((T, DIM), _F32)

    @functools.partial(
        pl.kernel,
        out_type=(row_t, row_t),
        mesh=mesh,
        scratch_types=[
            pltpu.VMEM((_SC_CH,), _I32),
            pltpu.VMEM((_SC_CH,), _I32),
            pltpu.VMEM((_SC_CH, DIM), _F32),
            pltpu.SemaphoreType.DMA,
        ],
    )
    def gather_k(eo_hbm, d1_hbm, d2_hbm, g1_hbm, g2_hbm, i1_v, i2_v, rows_v,
                 sem):
        wid = lax.axis_index("s") * 2 + lax.axis_index("c")
        base = wid * per_w

        def body(j, carry):
            b = base + j * _SC_CH
            pltpu.sync_copy(d1_hbm.at[pl.ds(b, _SC_CH)], i1_v)
            pltpu.sync_copy(d2_hbm.at[pl.ds(b, _SC_CH)], i2_v)
            pltpu.async_copy(eo_hbm.at[i1_v], rows_v, sem).wait()
            pltpu.sync_copy(rows_v, g1_hbm.at[pl.ds(b, _SC_CH)])
            pltpu.async_copy(eo_hbm.at[i2_v], rows_v, sem).wait()
            pltpu.sync_copy(rows_v, g2_hbm.at[pl.ds(b, _SC_CH)])
            return carry

        lax.fori_loop(0, per_w // _SC_CH, body, 0)

    return gather_k(eo, d1, d2)


def _ffn_body(offs_ref, grp_ref, blk_ref, xs_ref, w1_ref, w3_ref, w2_ref,
              out_ref, w1s_ref, w3s_ref):
    w = pl.program_id(0)
    g = grp_ref[w]
    st = offs_ref[g]
    en = offs_ref[g + 1]
    b = blk_ref[w]
    wprev = jnp.maximum(w - 1, 0)
    gcur = jnp.minimum(g, E - 1)
    gprev = jnp.minimum(grp_ref[wprev], E - 1)

    @pl.when((w == 0) | (gcur != gprev))
    def _():
        w1s_ref[...] = w1_ref[0].astype(_BF16)
        w3s_ref[...] = w3_ref[0].astype(_BF16)

    rid = b * BM + lax.broadcasted_iota(_I32, (BM, 1), 0)
    mask = (rid >= st) & (rid < en)
    x = xs_ref[...].astype(_BF16)
    a = _mm_t(x, w1s_ref[...])
    c3 = _mm_t(x, w3s_ref[...])
    h = ((a * jax.nn.sigmoid(a)) * c3).astype(_BF16)
    oe = _mm_t(h, w2_ref[0].astype(_BF16))
    contrib = jnp.where(mask, oe, 0.0)
    first = (w == 0) | (b != blk_ref[wprev])

    @pl.when(first)
    def _():
        out_ref[...] = contrib

    @pl.when(jnp.logical_not(first))
    def _():
        out_ref[...] += contrib


def _grouped_ffn(xs, w1, w3, w2, offs, grp, blk):
    def gmin(s_ref):
        return jnp.minimum(s_ref[0], E - 1)

    grid_spec = pltpu.PrefetchScalarGridSpec(
        num_scalar_prefetch=3,
        grid=(W,),
        in_specs=[
            pl.BlockSpec((BM, DIM), lambda w, o, g, b: (b[w], 0)),
            pl.BlockSpec((1, HID, DIM),
                         lambda w, o, g, b: (jnp.minimum(g[w], E - 1), 0, 0)),
            pl.BlockSpec((1, HID, DIM),
                         lambda w, o, g, b: (jnp.minimum(g[w], E - 1), 0, 0)),
            pl.BlockSpec((1, DIM, HID),
                         lambda w, o, g, b: (jnp.minimum(g[w], E - 1), 0, 0)),
        ],
        out_specs=pl.BlockSpec((BM, DIM), lambda w, o, g, b: (b[w], 0)),
        scratch_shapes=[
            pltpu.VMEM((HID, DIM), _BF16),
            pltpu.VMEM((HID, DIM), _BF16),
        ],
    )
    return pl.pallas_call(
        _ffn_body,
        grid_spec=grid_spec,
        out_shape=jax.ShapeDtypeStruct((TK, DIM), _F32),
        compiler_params=pltpu.CompilerParams(
            dimension_semantics=("arbitrary",),
            vmem_limit_bytes=120 * 1024 * 1024,
        ),
    )(offs, grp, blk, xs, w1, w3, w2)


def _shared_body(x_ref, ws1_ref, ws3_ref, ws2_ref, g1_ref, g2_ref,
                 s1_ref, s2_ref, out_ref, ws1s_ref, ws3s_ref, ws2s_ref):
    @pl.when(pl.program_id(0) == 0)
    def _():
        ws1s_ref[...] = ws1_ref[...].astype(_BF16)
        ws3s_ref[...] = ws3_ref[...].astype(_BF16)
        ws2s_ref[...] = ws2_ref[...].astype(_BF16)

    x = x_ref[...].astype(_BF16)
    a = _mm_t(x, ws1s_ref[...])
    c3 = _mm_t(x, ws3s_ref[...])
    h = ((a * jax.nn.sigmoid(a)) * c3).astype(_BF16)
    sh = _mm_t(h, ws2s_ref[...])
    out_ref[...] = sh + s1_ref[...] * g1_ref[...] + s2_ref[...] * g2_ref[...]


def _shared_combine(x, ws1, ws3, ws2, g1, g2, s1, s2):
    bm = 256
    grid = (T // bm,)
    row_spec = pl.BlockSpec((bm, DIM), lambda i: (i, 0))
    s_spec = pl.BlockSpec((bm, 1), lambda i: (i, 0))
    return pl.pallas_call(
        _shared_body,
        grid=grid,
        in_specs=[
            row_spec,
            pl.BlockSpec((HID, DIM), lambda i: (0, 0)),
            pl.BlockSpec((HID, DIM), lambda i: (0, 0)),
            pl.BlockSpec((DIM, HID), lambda i: (0, 0)),
            row_spec,
            row_spec,
            s_spec,
            s_spec,
        ],
        out_specs=row_spec,
        out_shape=jax.ShapeDtypeStruct((T, DIM), _F32),
        scratch_shapes=[
            pltpu.VMEM((HID, DIM), _BF16),
            pltpu.VMEM((HID, DIM), _BF16),
            pltpu.VMEM((DIM, HID), _BF16),
        ],
        compiler_params=pltpu.CompilerParams(
            vmem_limit_bytes=120 * 1024 * 1024,
        ),
    )(x, ws1, ws3, ws2, g1, g2, s1, s2)


def kernel(x, gate_w, w1, w2, w3, ws1, ws2, ws3, expert_bias):
    bias8 = jnp.broadcast_to(expert_bias[None, :], (8, E))
    e1, e2, s1, s2 = _router(x, gate_w, bias8)
    d1, d2, offs8, grp, blk = _sort(e1, e2)
    d1f = d1.reshape(TK // 2)
    d2f = d2.reshape(TK // 2)
    offs = offs8[0]
    xs = _sc_scatter(x, d1f, d2f)
    eo = _grouped_ffn(xs, w1, w3, w2, offs, grp.reshape(W), blk.reshape(W))
    g1, g2 = _sc_gather(eo, d1f, d2f)
    return _shared_combine(x, ws1, ws3, ws2, g1, g2, s1, s2)


# speedup vs baseline: 1.4218x; 1.4218x over previous
import functools

import jax
import jax.numpy as jnp
from jax import lax
from jax.experimental import pallas as pl
from jax.experimental.pallas import tpu as pltpu
from jax.experimental.pallas import tpu_sc as plsc

T = 4096
DIM = 2048
HID = 1024
E = 16
K = 2
TK = T * K
BM = 256
NB = TK // BM
W = NB + E

_F32 = jnp.float32
_BF16 = jnp.bfloat16
_I32 = jnp.int32


def _mm_t(a, b):
    return lax.dot_general(a, b, (((1,), (1,)), ((), ())),
                           preferred_element_type=_F32)


def _router_body(x_ref, gw_ref, bias_ref, e1_ref, e2_ref, s1_ref, s2_ref):
    x = x_ref[...]
    logits = _mm_t(x, gw_ref[...])
    scores = jax.nn.sigmoid(logits)
    biased = scores + bias_ref[0:1, :]
    iota_e = lax.broadcasted_iota(_I32, (1, E), 1)
    m1 = jnp.max(biased, axis=1, keepdims=True)
    a1 = jnp.min(jnp.where(biased == m1, iota_e, E), axis=1, keepdims=True)
    masked = jnp.where(iota_e == a1, -jnp.inf, biased)
    m2 = jnp.max(masked, axis=1, keepdims=True)
    a2 = jnp.min(jnp.where(masked == m2, iota_e, E), axis=1, keepdims=True)
    s1 = jnp.sum(jnp.where(iota_e == a1, scores, 0.0), axis=1, keepdims=True)
    s2 = jnp.sum(jnp.where(iota_e == a2, scores, 0.0), axis=1, keepdims=True)
    den = s1 + s2 + 1e-20
    e1_ref[...] = a1
    e2_ref[...] = a2
    s1_ref[...] = s1 / den
    s2_ref[...] = s2 / den


def _router(x, gate_w, bias8):
    bm = 1024
    grid = (T // bm,)
    out_shape = (
        jax.ShapeDtypeStruct((T, 1), _I32),
        jax.ShapeDtypeStruct((T, 1), _I32),
        jax.ShapeDtypeStruct((T, 1), _F32),
        jax.ShapeDtypeStruct((T, 1), _F32),
    )
    row_spec = pl.BlockSpec((bm, 1), lambda i: (i, 0))
    return pl.pallas_call(
        _router_body,
        grid=grid,
        in_specs=[
            pl.BlockSpec((bm, DIM), lambda i: (i, 0)),
            pl.BlockSpec((E, DIM), lambda i: (0, 0)),
            pl.BlockSpec((8, E), lambda i: (0, 0)),
        ],
        out_specs=(row_spec, row_spec, row_spec, row_spec),
        out_shape=out_shape,
    )(x, gate_w, bias8)


def _sort_body(e1_ref, e2_ref, d1_ref, d2_ref, offs_ref, grp_ref, blk_ref,
               grpw_ref, nxt_ref):
    iota_e = lax.broadcasted_iota(_I32, (1, E), 1)
    oh1 = (e1_ref[...] == iota_e).astype(_I32)
    oh2 = (e2_ref[...] == iota_e).astype(_I32)
    c = oh1 + oh2
    s = 1
    while s < T:
        c = c + jnp.concatenate(
            [jnp.zeros((s, E), _I32), c[: T - s]], axis=0)
        s *= 2
    total = c[T - 1: T, :]
    cnt_before = c - oh1 - oh2
    oi = total
    s = 1
    while s < E:
        oi = oi + jnp.concatenate(
            [jnp.zeros((1, s), _I32), oi[:, : E - s]], axis=1)
        s *= 2
    off_excl = oi - total
    d1_ref[...] = jnp.sum(oh1 * (off_excl + cnt_before), axis=1, keepdims=True)
    d2_ref[...] = jnp.sum(oh2 * (off_excl + cnt_before), axis=1, keepdims=True)
    offs = jnp.concatenate(
        [off_excl, jnp.full((1, 2), TK, _I32)], axis=1)
    offs_ref[...] = jnp.broadcast_to(offs, (8, E + 2))
    nz = total > 0
    fb = off_excl // BM
    lb = (jnp.maximum(oi, 1) - 1) // BM
    tiles = jnp.where(nz, lb - fb + 1, 0)
    cti = tiles
    s = 1
    while s < E:
        cti = cti + jnp.concatenate(
            [jnp.zeros((1, s), _I32), cti[:, : E - s]], axis=1)
        s *= 2
    cte = cti - tiles
    item = lax.broadcasted_iota(_I32, (W, 1), 0)
    gof = jnp.sum((cti <= item).astype(_I32), axis=1, keepdims=True)
    ohg = lax.broadcasted_iota(_I32, (W, E), 1) == gof
    blk = jnp.sum(jnp.where(ohg, fb - cte, 0), axis=1, keepdims=True) + item
    blk_ref[...] = jnp.where(gof >= E, NB - 1, blk)
    grp_ref[...] = jnp.minimum(gof, E)
    iota_l = lax.broadcasted_iota(_I32, (1, E), 1)
    last_real = jnp.max(jnp.where(nz, iota_l, 0), axis=1, keepdims=True)
    grpw_ref[...] = jnp.minimum(gof, last_real)
    nxt_start = jnp.sum(jnp.where(ohg, cti, 0), axis=1, keepdims=True)
    nxt_ref[...] = jnp.sum((cti <= nxt_start).astype(_I32), axis=1,
                           keepdims=True)


def _sort(e1, e2):
    wvec = jax.ShapeDtypeStruct((W, 1), _I32)
    out_shape = (
        jax.ShapeDtypeStruct((T, 1), _I32),
        jax.ShapeDtypeStruct((T, 1), _I32),
        jax.ShapeDtypeStruct((8, E + 2), _I32),
        wvec, wvec, wvec, wvec,
    )
    return pl.pallas_call(_sort_body, out_shape=out_shape)(e1, e2)


_SC_NW = 32
_SC_CH = 32


def _sc_scatter(x, d1, d2):
    mesh = plsc.VectorSubcoreMesh(core_axis_name="c", subcore_axis_name="s")
    per_w = T // _SC_NW

    @functools.partial(
        pl.kernel,
        out_type=jax.ShapeDtypeStruct((TK, DIM), _F32),
        mesh=mesh,
        scratch_types=[
            pltpu.VMEM((_SC_CH,), _I32),
            pltpu.VMEM((_SC_CH,), _I32),
            pltpu.VMEM((_SC_CH, DIM), _F32),
            pltpu.SemaphoreType.DMA,
            pltpu.SemaphoreType.DMA,
        ],
    )
    def scatter_k(x_hbm, d1_hbm, d2_hbm, out_hbm, i1_v, i2_v, rows_v,
                  sem1, sem2):
        wid = lax.axis_index("s") * 2 + lax.axis_index("c")
        base = wid * per_w

        def body(j, carry):
            b = base + j * _SC_CH
            pltpu.sync_copy(x_hbm.at[pl.ds(b, _SC_CH)], rows_v)
            pltpu.sync_copy(d1_hbm.at[pl.ds(b, _SC_CH)], i1_v)
            pltpu.sync_copy(d2_hbm.at[pl.ds(b, _SC_CH)], i2_v)
            c1 = pltpu.async_copy(rows_v, out_hbm.at[i1_v], sem1)
            c2 = pltpu.async_copy(rows_v, out_hbm.at[i2_v], sem2)
            c1.wait()
            c2.wait()
            return carry

        lax.fori_loop(0, per_w // _SC_CH, body, 0)

    return scatter_k(x, d1, d2)


def _sc_gather(eo, d1, d2):
    mesh = plsc.VectorSubcoreMesh(core_axis_name="c", subcore_axis_name="s")
    per_w = T // _SC_NW
    row_t = jax.ShapeDtypeStruct((T, DIM), _F32)

    @functools.partial(
        pl.kernel,
        out_type=(row_t, row_t),
        mesh=mesh,
        scratch_types=[
            pltpu.VMEM((_SC_CH,), _I32),
            pltpu.VMEM((_SC_CH,), _I32),
            pltpu.VMEM((_SC_CH, DIM), _F32),
            pltpu.SemaphoreType.DMA,
        ],
    )
    def gather_k(eo_hbm, d1_hbm, d2_hbm, g1_hbm, g2_hbm, i1_v, i2_v, rows_v,
                 sem):
        wid = lax.axis_index("s") * 2 + lax.axis_index("c")
        base = wid * per_w

        def body(j, carry):
            b = base + j * _SC_CH
            pltpu.sync_copy(d1_hbm.at[pl.ds(b, _SC_CH)], i1_v)
            pltpu.sync_copy(d2_hbm.at[pl.ds(b, _SC_CH)], i2_v)
            pltpu.async_copy(eo_hbm.at[i1_v], rows_v, sem).wait()
            pltpu.sync_copy(rows_v, g1_hbm.at[pl.ds(b, _SC_CH)])
            pltpu.async_copy(eo_hbm.at[i2_v], rows_v, sem).wait()
            pltpu.sync_copy(rows_v, g2_hbm.at[pl.ds(b, _SC_CH)])
            return carry

        lax.fori_loop(0, per_w // _SC_CH, body, 0)

    return gather_k(eo, d1, d2)


def _ffn_body(offs_ref, grp_ref, blk_ref, grpw_ref, nxt_ref,
              xs_ref, w1_ref, w3_ref, w2_ref, out_ref,
              w1f_ref, w3f_ref, w2f_ref, w1s_ref, w3s_ref, w2s_ref,
              sem1, sem3, sem2):
    w = pl.program_id(0)
    g = grp_ref[w]
    st = offs_ref[g]
    en = offs_ref[g + 1]
    b = blk_ref[w]
    wprev = jnp.maximum(w - 1, 0)
    gw = grpw_ref[w]

    @pl.when(w == 0)
    def _():
        pltpu.async_copy(w1_ref.at[gw], w1f_ref, sem1)
        pltpu.async_copy(w3_ref.at[gw], w3f_ref, sem3)
        pltpu.async_copy(w2_ref.at[gw], w2f_ref, sem2)

    @pl.when((w == 0) | (gw != grpw_ref[wprev]))
    def _():
        pltpu.make_async_copy(w1_ref.at[gw], w1f_ref, sem1).wait()
        pltpu.make_async_copy(w3_ref.at[gw], w3f_ref, sem3).wait()
        pltpu.make_async_copy(w2_ref.at[gw], w2f_ref, sem2).wait()
        w1s_ref[...] = w1f_ref[...].astype(_BF16)
        w3s_ref[...] = w3f_ref[...].astype(_BF16)
        w2s_ref[...] = w2f_ref[...].astype(_BF16)
        nx = nxt_ref[w]

        @pl.when(nx < E)
        def _():
            pltpu.async_copy(w1_ref.at[nx], w1f_ref, sem1)
            pltpu.async_copy(w3_ref.at[nx], w3f_ref, sem3)
            pltpu.async_copy(w2_ref.at[nx], w2f_ref, sem2)

    rid = b * BM + lax.broadcasted_iota(_I32, (BM, 1), 0)
    mask = (rid >= st) & (rid < en)
    x = xs_ref[...]
    a = _mm_t(x, w1s_ref[...])
    c3 = _mm_t(x, w3s_ref[...])
    h = ((a * jax.nn.sigmoid(a)) * c3).astype(_BF16)
    oe = _mm_t(h, w2s_ref[...])
    contrib = jnp.where(mask, oe, 0.0)
    first = (w == 0) | (b != blk_ref[wprev])

    @pl.when(first)
    def _():
        out_ref[...] = contrib

    @pl.when(jnp.logical_not(first))
    def _():
        out_ref[...] += contrib


def _grouped_ffn(xs, w1, w3, w2, offs, grp, blk, grpw, nxt):
    grid_spec = pltpu.PrefetchScalarGridSpec(
        num_scalar_prefetch=5,
        grid=(W,),
        in_specs=[
            pl.BlockSpec((BM, DIM), lambda w, o, g, b, gw, nx: (b[w], 0)),
            pl.BlockSpec(memory_space=pl.ANY),
            pl.BlockSpec(memory_space=pl.ANY),
            pl.BlockSpec(memory_space=pl.ANY),
        ],
        out_specs=pl.BlockSpec((BM, DIM), lambda w, o, g, b, gw, nx: (b[w], 0)),
        scratch_shapes=[
            pltpu.VMEM((HID, DIM), _F32),
            pltpu.VMEM((HID, DIM), _F32),
            pltpu.VMEM((DIM, HID), _F32),
            pltpu.VMEM((HID, DIM), _BF16),
            pltpu.VMEM((HID, DIM), _BF16),
            pltpu.VMEM((DIM, HID), _BF16),
            pltpu.SemaphoreType.DMA,
            pltpu.SemaphoreType.DMA,
            pltpu.SemaphoreType.DMA,
        ],
    )
    return pl.pallas_call(
        _ffn_body,
        grid_spec=grid_spec,
        out_shape=jax.ShapeDtypeStruct((TK, DIM), _F32),
        compiler_params=pltpu.CompilerParams(
            dimension_semantics=("arbitrary",),
            vmem_limit_bytes=120 * 1024 * 1024,
        ),
    )(offs, grp, blk, grpw, nxt, xs, w1, w3, w2)


def _shared_body(x_ref, ws1_ref, ws3_ref, ws2_ref, g1_ref, g2_ref,
                 s1_ref, s2_ref, out_ref, ws1s_ref, ws3s_ref, ws2s_ref):
    @pl.when(pl.program_id(0) == 0)
    def _():
        ws1s_ref[...] = ws1_ref[...].astype(_BF16)
        ws3s_ref[...] = ws3_ref[...].astype(_BF16)
        ws2s_ref[...] = ws2_ref[...].astype(_BF16)

    x = x_ref[...].astype(_BF16)
    a = _mm_t(x, ws1s_ref[...])
    c3 = _mm_t(x, ws3s_ref[...])
    h = ((a * jax.nn.sigmoid(a)) * c3).astype(_BF16)
    sh = _mm_t(h, ws2s_ref[...])
    out_ref[...] = sh + s1_ref[...] * g1_ref[...] + s2_ref[...] * g2_ref[...]


def _shared_combine(x, ws1, ws3, ws2, g1, g2, s1, s2):
    bm = 256
    grid = (T // bm,)
    row_spec = pl.BlockSpec((bm, DIM), lambda i: (i, 0))
    s_spec = pl.BlockSpec((bm, 1), lambda i: (i, 0))
    return pl.pallas_call(
        _shared_body,
        grid=grid,
        in_specs=[
            row_spec,
            pl.BlockSpec((HID, DIM), lambda i: (0, 0)),
            pl.BlockSpec((HID, DIM), lambda i: (0, 0)),
            pl.BlockSpec((DIM, HID), lambda i: (0, 0)),
            row_spec,
            row_spec,
            s_spec,
            s_spec,
        ],
        out_specs=row_spec,
        out_shape=jax.ShapeDtypeStruct((T, DIM), _F32),
        scratch_shapes=[
            pltpu.VMEM((HID, DIM), _BF16),
            pltpu.VMEM((HID, DIM), _BF16),
            pltpu.VMEM((DIM, HID), _BF16),
        ],
        compiler_params=pltpu.CompilerParams(
            vmem_limit_bytes=120 * 1024 * 1024,
        ),
    )(x, ws1, ws3, ws2, g1, g2, s1, s2)


def kernel(x, gate_w, w1, w2, w3, ws1, ws2, ws3, expert_bias):
    bias8 = jnp.broadcast_to(expert_bias[None, :], (8, E))
    e1, e2, s1, s2 = _router(x, gate_w, bias8)
    d1, d2, offs8, grp, blk, grpw, nxt = _sort(e1, e2)
    d1f = d1.reshape(TK // 2)
    d2f = d2.reshape(TK // 2)
    offs = offs8[0]
    xs = _sc_scatter(x, d1f, d2f)
    eo = _grouped_ffn(xs.astype(_BF16), w1, w3, w2, offs,
                      grp.reshape(W), blk.reshape(W),
                      grpw.reshape(W), nxt.reshape(W))
    g1, g2 = _sc_gather(eo, d1f, d2f)
    return _shared_combine(x, ws1, ws3, ws2, g1, g2, s1, s2)


# speedup vs baseline: 1.4875x; 1.0462x over previous
import functools

import jax
import jax.numpy as jnp
from jax import lax
from jax.experimental import pallas as pl
from jax.experimental.pallas import tpu as pltpu
from jax.experimental.pallas import tpu_sc as plsc

T = 4096
DIM = 2048
HID = 1024
E = 16
K = 2
TK = T * K
BM = 256
NB = TK // BM
W = NB + E

_F32 = jnp.float32
_BF16 = jnp.bfloat16
_I32 = jnp.int32


def _mm_t(a, b):
    return lax.dot_general(a, b, (((1,), (1,)), ((), ())),
                           preferred_element_type=_F32)


def _router_body(x_ref, gw_ref, bias_ref, e1_ref, e2_ref, s1_ref, s2_ref):
    x = x_ref[...]
    logits = _mm_t(x, gw_ref[...])
    scores = jax.nn.sigmoid(logits)
    biased = scores + bias_ref[0:1, :]
    iota_e = lax.broadcasted_iota(_I32, (1, E), 1)
    m1 = jnp.max(biased, axis=1, keepdims=True)
    a1 = jnp.min(jnp.where(biased == m1, iota_e, E), axis=1, keepdims=True)
    masked = jnp.where(iota_e == a1, -jnp.inf, biased)
    m2 = jnp.max(masked, axis=1, keepdims=True)
    a2 = jnp.min(jnp.where(masked == m2, iota_e, E), axis=1, keepdims=True)
    s1 = jnp.sum(jnp.where(iota_e == a1, scores, 0.0), axis=1, keepdims=True)
    s2 = jnp.sum(jnp.where(iota_e == a2, scores, 0.0), axis=1, keepdims=True)
    den = s1 + s2 + 1e-20
    e1_ref[...] = a1
    e2_ref[...] = a2
    s1_ref[...] = s1 / den
    s2_ref[...] = s2 / den


def _router(x, gate_w, bias8):
    bm = 1024
    grid = (T // bm,)
    out_shape = (
        jax.ShapeDtypeStruct((T, 1), _I32),
        jax.ShapeDtypeStruct((T, 1), _I32),
        jax.ShapeDtypeStruct((T, 1), _F32),
        jax.ShapeDtypeStruct((T, 1), _F32),
    )
    row_spec = pl.BlockSpec((bm, 1), lambda i: (i, 0))
    return pl.pallas_call(
        _router_body,
        grid=grid,
        in_specs=[
            pl.BlockSpec((bm, DIM), lambda i: (i, 0)),
            pl.BlockSpec((E, DIM), lambda i: (0, 0)),
            pl.BlockSpec((8, E), lambda i: (0, 0)),
        ],
        out_specs=(row_spec, row_spec, row_spec, row_spec),
        out_shape=out_shape,
    )(x, gate_w, bias8)


def _sort_body(e1_ref, e2_ref, d1_ref, d2_ref, offs_ref, grp_ref, blk_ref,
               grpw_ref, nxt_ref):
    iota_e = lax.broadcasted_iota(_I32, (1, E), 1)
    oh1 = (e1_ref[...] == iota_e).astype(_I32)
    oh2 = (e2_ref[...] == iota_e).astype(_I32)
    c = oh1 + oh2
    s = 1
    while s < T:
        c = c + jnp.concatenate(
            [jnp.zeros((s, E), _I32), c[: T - s]], axis=0)
        s *= 2
    total = c[T - 1: T, :]
    cnt_before = c - oh1 - oh2
    oi = total
    s = 1
    while s < E:
        oi = oi + jnp.concatenate(
            [jnp.zeros((1, s), _I32), oi[:, : E - s]], axis=1)
        s *= 2
    off_excl = oi - total
    d1_ref[...] = jnp.sum(oh1 * (off_excl + cnt_before), axis=1, keepdims=True)
    d2_ref[...] = jnp.sum(oh2 * (off_excl + cnt_before), axis=1, keepdims=True)
    offs = jnp.concatenate(
        [off_excl, jnp.full((1, 2), TK, _I32)], axis=1)
    offs_ref[...] = jnp.broadcast_to(offs, (8, E + 2))
    nz = total > 0
    fb = off_excl // BM
    lb = (jnp.maximum(oi, 1) - 1) // BM
    tiles = jnp.where(nz, lb - fb + 1, 0)
    cti = tiles
    s = 1
    while s < E:
        cti = cti + jnp.concatenate(
            [jnp.zeros((1, s), _I32), cti[:, : E - s]], axis=1)
        s *= 2
    cte = cti - tiles
    item = lax.broadcasted_iota(_I32, (W, 1), 0)
    gof = jnp.sum((cti <= item).astype(_I32), axis=1, keepdims=True)
    ohg = lax.broadcasted_iota(_I32, (W, E), 1) == gof
    blk = jnp.sum(jnp.where(ohg, fb - cte, 0), axis=1, keepdims=True) + item
    blk_ref[...] = jnp.where(gof >= E, NB - 1, blk)
    grp_ref[...] = jnp.minimum(gof, E)
    iota_l = lax.broadcasted_iota(_I32, (1, E), 1)
    last_real = jnp.max(jnp.where(nz, iota_l, 0), axis=1, keepdims=True)
    grpw_ref[...] = jnp.minimum(gof, last_real)
    nxt_start = jnp.sum(jnp.where(ohg, cti, 0), axis=1, keepdims=True)
    nxt_ref[...] = jnp.sum((cti <= nxt_start).astype(_I32), axis=1,
                           keepdims=True)


def _sort(e1, e2):
    wvec = jax.ShapeDtypeStruct((W, 1), _I32)
    out_shape = (
        jax.ShapeDtypeStruct((T, 1), _I32),
        jax.ShapeDtypeStruct((T, 1), _I32),
        jax.ShapeDtypeStruct((8, E + 2), _I32),
        wvec, wvec, wvec, wvec,
    )
    return pl.pallas_call(_sort_body, out_shape=out_shape)(e1, e2)


_SC_NW = 32
_SC_CH = 32


def _sc_scatter(x, d1, d2):
    mesh = plsc.VectorSubcoreMesh(core_axis_name="c", subcore_axis_name="s")
    per_w = T // _SC_NW

    @functools.partial(
        pl.kernel,
        out_type=jax.ShapeDtypeStruct((TK, DIM), _F32),
        mesh=mesh,
        scratch_types=[
            pltpu.VMEM((_SC_CH,), _I32),
            pltpu.VMEM((_SC_CH,), _I32),
            pltpu.VMEM((_SC_CH, DIM), _F32),
            pltpu.SemaphoreType.DMA,
            pltpu.SemaphoreType.DMA,
        ],
    )
    def scatter_k(x_hbm, d1_hbm, d2_hbm, out_hbm, i1_v, i2_v, rows_v,
                  sem1, sem2):
        wid = lax.axis_index("s") * 2 + lax.axis_index("c")
        base = wid * per_w

        def body(j, carry):
            b = base + j * _SC_CH
            pltpu.sync_copy(x_hbm.at[pl.ds(b, _SC_CH)], rows_v)
            pltpu.sync_copy(d1_hbm.at[pl.ds(b, _SC_CH)], i1_v)
            pltpu.sync_copy(d2_hbm.at[pl.ds(b, _SC_CH)], i2_v)
            c1 = pltpu.async_copy(rows_v, out_hbm.at[i1_v], sem1)
            c2 = pltpu.async_copy(rows_v, out_hbm.at[i2_v], sem2)
            c1.wait()
            c2.wait()
            return carry

        lax.fori_loop(0, per_w // _SC_CH, body, 0)

    return scatter_k(x, d1, d2)


def _sc_gather(eo, d1, d2):
    mesh = plsc.VectorSubcoreMesh(core_axis_name="c", subcore_axis_name="s")
    per_w = T // _SC_NW
    row_t = jax.ShapeDtypeStruct((T, DIM), _F32)

    @functools.partial(
        pl.kernel,
        out_type=(row_t, row_t),
        mesh=mesh,
        scratch_types=[
            pltpu.VMEM((_SC_CH,), _I32),
            pltpu.VMEM((_SC_CH,), _I32),
            pltpu.VMEM((_SC_CH, DIM), _F32),
            pltpu.SemaphoreType.DMA,
        ],
    )
    def gather_k(eo_hbm, d1_hbm, d2_hbm, g1_hbm, g2_hbm, i1_v, i2_v, rows_v,
                 sem):
        wid = lax.axis_index("s") * 2 + lax.axis_index("c")
        base = wid * per_w

        def body(j, carry):
            b = base + j * _SC_CH
            pltpu.sync_copy(d1_hbm.at[pl.ds(b, _SC_CH)], i1_v)
            pltpu.sync_copy(d2_hbm.at[pl.ds(b, _SC_CH)], i2_v)
            pltpu.async_copy(eo_hbm.at[i1_v], rows_v, sem).wait()
            pltpu.sync_copy(rows_v, g1_hbm.at[pl.ds(b, _SC_CH)])
            pltpu.async_copy(eo_hbm.at[i2_v], rows_v, sem).wait()
            pltpu.sync_copy(rows_v, g2_hbm.at[pl.ds(b, _SC_CH)])
            return carry

        lax.fori_loop(0, per_w // _SC_CH, body, 0)

    return gather_k(eo, d1, d2)


def _ffn_body(offs_ref, grp_ref, blk_ref, grpw_ref, nxt_ref,
              xs_ref, w1_ref, w3_ref, w2_ref, out_ref,
              w1f_ref, w3f_ref, w2f_ref, w1s_ref, w3s_ref, w2s_ref,
              sem1, sem3, sem2):
    w = pl.program_id(0)
    g = grp_ref[w]
    st = offs_ref[g]
    en = offs_ref[g + 1]
    b = blk_ref[w]
    wprev = jnp.maximum(w - 1, 0)
    gw = grpw_ref[w]

    @pl.when(w == 0)
    def _():
        pltpu.async_copy(w1_ref.at[gw], w1f_ref, sem1)
        pltpu.async_copy(w3_ref.at[gw], w3f_ref, sem3)
        pltpu.async_copy(w2_ref.at[gw], w2f_ref, sem2)

    @pl.when((w == 0) | (gw != grpw_ref[wprev]))
    def _():
        pltpu.make_async_copy(w1_ref.at[gw], w1f_ref, sem1).wait()
        pltpu.make_async_copy(w3_ref.at[gw], w3f_ref, sem3).wait()
        pltpu.make_async_copy(w2_ref.at[gw], w2f_ref, sem2).wait()
        w1s_ref[...] = w1f_ref[...].astype(_BF16)
        w3s_ref[...] = w3f_ref[...].astype(_BF16)
        w2s_ref[...] = w2f_ref[...].astype(_BF16)
        nx = nxt_ref[w]

        @pl.when(nx < E)
        def _():
            pltpu.async_copy(w1_ref.at[nx], w1f_ref, sem1)
            pltpu.async_copy(w3_ref.at[nx], w3f_ref, sem3)
            pltpu.async_copy(w2_ref.at[nx], w2f_ref, sem2)

    @pl.when(st < en)
    def _():
        rid = b * BM + lax.broadcasted_iota(_I32, (BM, 1), 0)
        mask = (rid >= st) & (rid < en)
        x = xs_ref[...].astype(_BF16)
        a = _mm_t(x, w1s_ref[...])
        c3 = _mm_t(x, w3s_ref[...])
        h = ((a * jax.nn.sigmoid(a)) * c3).astype(_BF16)
        oe = _mm_t(h, w2s_ref[...])
        contrib = jnp.where(mask, oe, 0.0)
        first = (w == 0) | (b != blk_ref[wprev])

        @pl.when(first)
        def _():
            out_ref[...] = contrib

        @pl.when(jnp.logical_not(first))
        def _():
            out_ref[...] += contrib


def _grouped_ffn(xs, w1, w3, w2, offs, grp, blk, grpw, nxt):
    grid_spec = pltpu.PrefetchScalarGridSpec(
        num_scalar_prefetch=5,
        grid=(W,),
        in_specs=[
            pl.BlockSpec((BM, DIM), lambda w, o, g, b, gw, nx: (b[w], 0)),
            pl.BlockSpec(memory_space=pl.ANY),
            pl.BlockSpec(memory_space=pl.ANY),
            pl.BlockSpec(memory_space=pl.ANY),
        ],
        out_specs=pl.BlockSpec((BM, DIM), lambda w, o, g, b, gw, nx: (b[w], 0)),
        scratch_shapes=[
            pltpu.VMEM((HID, DIM), _F32),
            pltpu.VMEM((HID, DIM), _F32),
            pltpu.VMEM((DIM, HID), _F32),
            pltpu.VMEM((HID, DIM), _BF16),
            pltpu.VMEM((HID, DIM), _BF16),
            pltpu.VMEM((DIM, HID), _BF16),
            pltpu.SemaphoreType.DMA,
            pltpu.SemaphoreType.DMA,
            pltpu.SemaphoreType.DMA,
        ],
    )
    return pl.pallas_call(
        _ffn_body,
        grid_spec=grid_spec,
        out_shape=jax.ShapeDtypeStruct((TK, DIM), _F32),
        compiler_params=pltpu.CompilerParams(
            dimension_semantics=("arbitrary",),
            vmem_limit_bytes=120 * 1024 * 1024,
        ),
    )(offs, grp, blk, grpw, nxt, xs, w1, w3, w2)


def _shared_body(x_ref, ws1_ref, ws3_ref, ws2_ref, g1_ref, g2_ref,
                 s1_ref, s2_ref, out_ref, ws1s_ref, ws3s_ref, ws2s_ref):
    @pl.when(pl.program_id(0) == 0)
    def _():
        ws1s_ref[...] = ws1_ref[...].astype(_BF16)
        ws3s_ref[...] = ws3_ref[...].astype(_BF16)
        ws2s_ref[...] = ws2_ref[...].astype(_BF16)

    x = x_ref[...].astype(_BF16)
    a = _mm_t(x, ws1s_ref[...])
    c3 = _mm_t(x, ws3s_ref[...])
    h = ((a * jax.nn.sigmoid(a)) * c3).astype(_BF16)
    sh = _mm_t(h, ws2s_ref[...])
    out_ref[...] = sh + s1_ref[...] * g1_ref[...] + s2_ref[...] * g2_ref[...]


def _shared_combine(x, ws1, ws3, ws2, g1, g2, s1, s2):
    bm = 256
    grid = (T // bm,)
    row_spec = pl.BlockSpec((bm, DIM), lambda i: (i, 0))
    s_spec = pl.BlockSpec((bm, 1), lambda i: (i, 0))
    return pl.pallas_call(
        _shared_body,
        grid=grid,
        in_specs=[
            row_spec,
            pl.BlockSpec((HID, DIM), lambda i: (0, 0)),
            pl.BlockSpec((HID, DIM), lambda i: (0, 0)),
            pl.BlockSpec((DIM, HID), lambda i: (0, 0)),
            row_spec,
            row_spec,
            s_spec,
            s_spec,
        ],
        out_specs=row_spec,
        out_shape=jax.ShapeDtypeStruct((T, DIM), _F32),
        scratch_shapes=[
            pltpu.VMEM((HID, DIM), _BF16),
            pltpu.VMEM((HID, DIM), _BF16),
            pltpu.VMEM((DIM, HID), _BF16),
        ],
        compiler_params=pltpu.CompilerParams(
            vmem_limit_bytes=120 * 1024 * 1024,
        ),
    )(x, ws1, ws3, ws2, g1, g2, s1, s2)


def kernel(x, gate_w, w1, w2, w3, ws1, ws2, ws3, expert_bias):
    bias8 = jnp.broadcast_to(expert_bias[None, :], (8, E))
    e1, e2, s1, s2 = _router(x, gate_w, bias8)
    d1, d2, offs8, grp, blk, grpw, nxt = _sort(e1, e2)
    d1f = d1.reshape(TK // 2)
    d2f = d2.reshape(TK // 2)
    offs = offs8[0]
    xs = _sc_scatter(x, d1f, d2f)
    eo = _grouped_ffn(xs, w1, w3, w2, offs,
                      grp.reshape(W), blk.reshape(W),
                      grpw.reshape(W), nxt.reshape(W))
    g1, g2 = _sc_gather(eo, d1f, d2f)
    return _shared_combine(x, ws1, ws3, ws2, g1, g2, s1, s2)


# speedup vs baseline: 1.5123x; 1.0167x over previous
import functools

import jax
import jax.numpy as jnp
from jax import lax
from jax.experimental import pallas as pl
from jax.experimental.pallas import tpu as pltpu
from jax.experimental.pallas import tpu_sc as plsc

T = 4096
DIM = 2048
HID = 1024
E = 16
K = 2
TK = T * K
BM = 256
NB = TK // BM
W = NB + E

_F32 = jnp.float32
_BF16 = jnp.bfloat16
_I32 = jnp.int32


def _mm_t(a, b):
    return lax.dot_general(a, b, (((1,), (1,)), ((), ())),
                           preferred_element_type=_F32)


def _router_body(x_ref, gw_ref, bias_ref, e1_ref, e2_ref, s1_ref, s2_ref):
    x = x_ref[...]
    logits = _mm_t(x, gw_ref[...])
    scores = jax.nn.sigmoid(logits)
    biased = scores + bias_ref[0:1, :]
    iota_e = lax.broadcasted_iota(_I32, (1, E), 1)
    m1 = jnp.max(biased, axis=1, keepdims=True)
    a1 = jnp.min(jnp.where(biased == m1, iota_e, E), axis=1, keepdims=True)
    masked = jnp.where(iota_e == a1, -jnp.inf, biased)
    m2 = jnp.max(masked, axis=1, keepdims=True)
    a2 = jnp.min(jnp.where(masked == m2, iota_e, E), axis=1, keepdims=True)
    s1 = jnp.sum(jnp.where(iota_e == a1, scores, 0.0), axis=1, keepdims=True)
    s2 = jnp.sum(jnp.where(iota_e == a2, scores, 0.0), axis=1, keepdims=True)
    den = s1 + s2 + 1e-20
    e1_ref[...] = a1
    e2_ref[...] = a2
    s1_ref[...] = s1 / den
    s2_ref[...] = s2 / den


def _router(x, gate_w, bias8):
    bm = 1024
    grid = (T // bm,)
    out_shape = (
        jax.ShapeDtypeStruct((T, 1), _I32),
        jax.ShapeDtypeStruct((T, 1), _I32),
        jax.ShapeDtypeStruct((T, 1), _F32),
        jax.ShapeDtypeStruct((T, 1), _F32),
    )
    row_spec = pl.BlockSpec((bm, 1), lambda i: (i, 0))
    return pl.pallas_call(
        _router_body,
        grid=grid,
        in_specs=[
            pl.BlockSpec((bm, DIM), lambda i: (i, 0)),
            pl.BlockSpec((E, DIM), lambda i: (0, 0)),
            pl.BlockSpec((8, E), lambda i: (0, 0)),
        ],
        out_specs=(row_spec, row_spec, row_spec, row_spec),
        out_shape=out_shape,
    )(x, gate_w, bias8)


def _sort_body(e1_ref, e2_ref, d1_ref, d2_ref, offs_ref, grp_ref, blk_ref,
               grpw_ref, nxt_ref):
    iota_e = lax.broadcasted_iota(_I32, (1, E), 1)
    oh1 = (e1_ref[...] == iota_e).astype(_I32)
    oh2 = (e2_ref[...] == iota_e).astype(_I32)
    c = oh1 + oh2
    s = 1
    while s < T:
        c = c + jnp.concatenate(
            [jnp.zeros((s, E), _I32), c[: T - s]], axis=0)
        s *= 2
    total = c[T - 1: T, :]
    cnt_before = c - oh1 - oh2
    oi = total
    s = 1
    while s < E:
        oi = oi + jnp.concatenate(
            [jnp.zeros((1, s), _I32), oi[:, : E - s]], axis=1)
        s *= 2
    off_excl = oi - total
    d1_ref[...] = jnp.sum(oh1 * (off_excl + cnt_before), axis=1, keepdims=True)
    d2_ref[...] = jnp.sum(oh2 * (off_excl + cnt_before), axis=1, keepdims=True)
    offs = jnp.concatenate(
        [off_excl, jnp.full((1, 2), TK, _I32)], axis=1)
    offs_ref[...] = jnp.broadcast_to(offs, (8, E + 2))
    nz = total > 0
    fb = off_excl // BM
    lb = (jnp.maximum(oi, 1) - 1) // BM
    tiles = jnp.where(nz, lb - fb + 1, 0)
    cti = tiles
    s = 1
    while s < E:
        cti = cti + jnp.concatenate(
            [jnp.zeros((1, s), _I32), cti[:, : E - s]], axis=1)
        s *= 2
    cte = cti - tiles
    item = lax.broadcasted_iota(_I32, (W, 1), 0)
    gof = jnp.sum((cti <= item).astype(_I32), axis=1, keepdims=True)
    ohg = lax.broadcasted_iota(_I32, (W, E), 1) == gof
    blk = jnp.sum(jnp.where(ohg, fb - cte, 0), axis=1, keepdims=True) + item
    blk_ref[...] = jnp.where(gof >= E, NB - 1, blk)
    grp_ref[...] = jnp.minimum(gof, E)
    iota_l = lax.broadcasted_iota(_I32, (1, E), 1)
    last_real = jnp.max(jnp.where(nz, iota_l, 0), axis=1, keepdims=True)
    grpw_ref[...] = jnp.minimum(gof, last_real)
    nxt_start = jnp.sum(jnp.where(ohg, cti, 0), axis=1, keepdims=True)
    nxt_ref[...] = jnp.sum((cti <= nxt_start).astype(_I32), axis=1,
                           keepdims=True)


def _sort(e1, e2):
    wvec = jax.ShapeDtypeStruct((W, 1), _I32)
    out_shape = (
        jax.ShapeDtypeStruct((T, 1), _I32),
        jax.ShapeDtypeStruct((T, 1), _I32),
        jax.ShapeDtypeStruct((8, E + 2), _I32),
        wvec, wvec, wvec, wvec,
    )
    return pl.pallas_call(_sort_body, out_shape=out_shape)(e1, e2)


_SC_NW = 32
_SC_CH = 32


def _sc_scatter(x, d1, d2):
    mesh = plsc.VectorSubcoreMesh(core_axis_name="c", subcore_axis_name="s")
    per_w = T // _SC_NW

    @functools.partial(
        pl.kernel,
        out_type=jax.ShapeDtypeStruct((TK, DIM), _F32),
        mesh=mesh,
        scratch_types=[
            pltpu.VMEM((_SC_CH,), _I32),
            pltpu.VMEM((_SC_CH,), _I32),
            pltpu.VMEM((_SC_CH, DIM), _F32),
            pltpu.SemaphoreType.DMA,
            pltpu.SemaphoreType.DMA,
        ],
    )
    def scatter_k(x_hbm, d1_hbm, d2_hbm, out_hbm, i1_v, i2_v, rows_v,
                  sem1, sem2):
        wid = lax.axis_index("s") * 2 + lax.axis_index("c")
        base = wid * per_w

        def body(j, carry):
            b = base + j * _SC_CH
            pltpu.sync_copy(x_hbm.at[pl.ds(b, _SC_CH)], rows_v)
            pltpu.sync_copy(d1_hbm.at[pl.ds(b, _SC_CH)], i1_v)
            pltpu.sync_copy(d2_hbm.at[pl.ds(b, _SC_CH)], i2_v)
            c1 = pltpu.async_copy(rows_v, out_hbm.at[i1_v], sem1)
            c2 = pltpu.async_copy(rows_v, out_hbm.at[i2_v], sem2)
            c1.wait()
            c2.wait()
            return carry

        lax.fori_loop(0, per_w // _SC_CH, body, 0)

    return scatter_k(x, d1, d2)


def _sc_gather(eo, d1, d2):
    mesh = plsc.VectorSubcoreMesh(core_axis_name="c", subcore_axis_name="s")
    per_w = T // _SC_NW
    row_t = jax.ShapeDtypeStruct((T, DIM), _F32)

    @functools.partial(
        pl.kernel,
        out_type=(row_t, row_t),
        mesh=mesh,
        scratch_types=[
            pltpu.VMEM((_SC_CH,), _I32),
            pltpu.VMEM((_SC_CH,), _I32),
            pltpu.VMEM((_SC_CH, DIM), _F32),
            pltpu.SemaphoreType.DMA,
        ],
    )
    def gather_k(eo_hbm, d1_hbm, d2_hbm, g1_hbm, g2_hbm, i1_v, i2_v, rows_v,
                 sem):
        wid = lax.axis_index("s") * 2 + lax.axis_index("c")
        base = wid * per_w

        def body(j, carry):
            b = base + j * _SC_CH
            pltpu.sync_copy(d1_hbm.at[pl.ds(b, _SC_CH)], i1_v)
            pltpu.sync_copy(d2_hbm.at[pl.ds(b, _SC_CH)], i2_v)
            pltpu.async_copy(eo_hbm.at[i1_v], rows_v, sem).wait()
            pltpu.sync_copy(rows_v, g1_hbm.at[pl.ds(b, _SC_CH)])
            pltpu.async_copy(eo_hbm.at[i2_v], rows_v, sem).wait()
            pltpu.sync_copy(rows_v, g2_hbm.at[pl.ds(b, _SC_CH)])
            return carry

        lax.fori_loop(0, per_w // _SC_CH, body, 0)

    return gather_k(eo, d1, d2)


def _ffn_body(offs_ref, grp_ref, blk_ref, grpw_ref, nxt_ref,
              xs_ref, w1_ref, w3_ref, w2_ref, out_ref,
              w1f_ref, w3f_ref, w2f_ref, w1s_ref, w3s_ref, w2s_ref,
              sem1, sem3, sem2):
    w = pl.program_id(0)
    g = grp_ref[w]
    st = offs_ref[g]
    en = offs_ref[g + 1]
    b = blk_ref[w]
    wprev = jnp.maximum(w - 1, 0)
    gw = grpw_ref[w]

    @pl.when(w == 0)
    def _():
        pltpu.async_copy(w1_ref.at[gw], w1f_ref, sem1)
        pltpu.async_copy(w3_ref.at[gw], w3f_ref, sem3)
        pltpu.async_copy(w2_ref.at[gw], w2f_ref, sem2)

    @pl.when((w == 0) | (gw != grpw_ref[wprev]))
    def _():
        pltpu.make_async_copy(w1_ref.at[gw], w1f_ref, sem1).wait()
        pltpu.make_async_copy(w3_ref.at[gw], w3f_ref, sem3).wait()
        pltpu.make_async_copy(w2_ref.at[gw], w2f_ref, sem2).wait()
        w1s_ref[...] = w1f_ref[...].astype(_BF16)
        w3s_ref[...] = w3f_ref[...].astype(_BF16)
        w2s_ref[...] = w2f_ref[...].astype(_BF16)
        nx = nxt_ref[w]

        @pl.when(nx < E)
        def _():
            pltpu.async_copy(w1_ref.at[nx], w1f_ref, sem1)
            pltpu.async_copy(w3_ref.at[nx], w3f_ref, sem3)
            pltpu.async_copy(w2_ref.at[nx], w2f_ref, sem2)

    @pl.when(st < en)
    def _():
        rid = b * BM + lax.broadcasted_iota(_I32, (BM, 1), 0)
        mask = (rid >= st) & (rid < en)
        x = xs_ref[...].astype(_BF16)
        a = _mm_t(x, w1s_ref[...])
        c3 = _mm_t(x, w3s_ref[...])
        h = ((a * jax.nn.sigmoid(a)) * c3).astype(_BF16)
        oe = _mm_t(h, w2s_ref[...])
        contrib = jnp.where(mask, oe, 0.0)
        first = (w == 0) | (b != blk_ref[wprev])

        @pl.when(first)
        def _():
            out_ref[...] = contrib

        @pl.when(jnp.logical_not(first))
        def _():
            out_ref[...] += contrib


def _grouped_ffn(xs, w1, w3, w2, offs, grp, blk, grpw, nxt):
    grid_spec = pltpu.PrefetchScalarGridSpec(
        num_scalar_prefetch=5,
        grid=(W,),
        in_specs=[
            pl.BlockSpec((BM, DIM), lambda w, o, g, b, gw, nx: (b[w], 0)),
            pl.BlockSpec(memory_space=pl.ANY),
            pl.BlockSpec(memory_space=pl.ANY),
            pl.BlockSpec(memory_space=pl.ANY),
        ],
        out_specs=pl.BlockSpec((BM, DIM), lambda w, o, g, b, gw, nx: (b[w], 0)),
        scratch_shapes=[
            pltpu.VMEM((HID, DIM), _F32),
            pltpu.VMEM((HID, DIM), _F32),
            pltpu.VMEM((DIM, HID), _F32),
            pltpu.VMEM((HID, DIM), _BF16),
            pltpu.VMEM((HID, DIM), _BF16),
            pltpu.VMEM((DIM, HID), _BF16),
            pltpu.SemaphoreType.DMA,
            pltpu.SemaphoreType.DMA,
            pltpu.SemaphoreType.DMA,
        ],
    )
    return pl.pallas_call(
        _ffn_body,
        grid_spec=grid_spec,
        out_shape=jax.ShapeDtypeStruct((TK, DIM), _F32),
        compiler_params=pltpu.CompilerParams(
            dimension_semantics=("arbitrary",),
            vmem_limit_bytes=120 * 1024 * 1024,
        ),
    )(offs, grp, blk, grpw, nxt, xs, w1, w3, w2)


def _shared_body(x_ref, ws1_ref, ws3_ref, ws2_ref, out_ref,
                 ws1s_ref, ws3s_ref, ws2s_ref):
    @pl.when(pl.program_id(0) == 0)
    def _():
        ws1s_ref[...] = ws1_ref[...].astype(_BF16)
        ws3s_ref[...] = ws3_ref[...].astype(_BF16)
        ws2s_ref[...] = ws2_ref[...].astype(_BF16)

    x = x_ref[...].astype(_BF16)
    a = _mm_t(x, ws1s_ref[...])
    c3 = _mm_t(x, ws3s_ref[...])
    h = ((a * jax.nn.sigmoid(a)) * c3).astype(_BF16)
    out_ref[...] = _mm_t(h, ws2s_ref[...])


def _shared_ffn(x, ws1, ws3, ws2):
    bm = 256
    grid = (T // bm,)
    row_spec = pl.BlockSpec((bm, DIM), lambda i: (i, 0))
    return pl.pallas_call(
        _shared_body,
        grid=grid,
        in_specs=[
            row_spec,
            pl.BlockSpec((HID, DIM), lambda i: (0, 0)),
            pl.BlockSpec((HID, DIM), lambda i: (0, 0)),
            pl.BlockSpec((DIM, HID), lambda i: (0, 0)),
        ],
        out_specs=row_spec,
        out_shape=jax.ShapeDtypeStruct((T, DIM), _F32),
        scratch_shapes=[
            pltpu.VMEM((HID, DIM), _BF16),
            pltpu.VMEM((HID, DIM), _BF16),
            pltpu.VMEM((DIM, HID), _BF16),
        ],
        compiler_params=pltpu.CompilerParams(
            vmem_limit_bytes=120 * 1024 * 1024,
        ),
    )(x, ws1, ws3, ws2)


def _combine_body(sh_ref, g1_ref, g2_ref, s1_ref, s2_ref, out_ref):
    out_ref[...] = (sh_ref[...] + s1_ref[...] * g1_ref[...]
                    + s2_ref[...] * g2_ref[...])


def _combine(sh, g1, g2, s1, s2):
    bm = 512
    grid = (T // bm,)
    row_spec = pl.BlockSpec((bm, DIM), lambda i: (i, 0))
    s_spec = pl.BlockSpec((bm, 1), lambda i: (i, 0))
    return pl.pallas_call(
        _combine_body,
        grid=grid,
        in_specs=[row_spec, row_spec, row_spec, s_spec, s_spec],
        out_specs=row_spec,
        out_shape=jax.ShapeDtypeStruct((T, DIM), _F32),
    )(sh, g1, g2, s1, s2)


def kernel(x, gate_w, w1, w2, w3, ws1, ws2, ws3, expert_bias):
    bias8 = jnp.broadcast_to(expert_bias[None, :], (8, E))
    e1, e2, s1, s2 = _router(x, gate_w, bias8)
    d1, d2, offs8, grp, blk, grpw, nxt = _sort(e1, e2)
    d1f = d1.reshape(TK // 2)
    d2f = d2.reshape(TK // 2)
    offs = offs8[0]
    xs = _sc_scatter(x, d1f, d2f)
    eo = _grouped_ffn(xs, w1, w3, w2, offs,
                      grp.reshape(W), blk.reshape(W),
                      grpw.reshape(W), nxt.reshape(W))
    g1, g2 = _sc_gather(eo, d1f, d2f)
    sh = _shared_ffn(x, ws1, ws3, ws2)
    return _combine(sh, g1, g2, s1, s2)


# speedup vs baseline: 1.5280x; 1.0103x over previous
import functools

import jax
import jax.numpy as jnp
from jax import lax
from jax.experimental import pallas as pl
from jax.experimental.pallas import tpu as pltpu
from jax.experimental.pallas import tpu_sc as plsc

T = 4096
DIM = 2048
HID = 1024
E = 16
K = 2
TK = T * K
BM = 256
NB = TK // BM
W = NB + E

_F32 = jnp.float32
_BF16 = jnp.bfloat16
_I32 = jnp.int32


def _mm_t(a, b):
    return lax.dot_general(a, b, (((1,), (1,)), ((), ())),
                           preferred_element_type=_F32)


def _router_body(x_ref, gw_ref, bias_ref, e1_ref, e2_ref, s1_ref, s2_ref):
    x = x_ref[...]
    logits = _mm_t(x, gw_ref[...])
    scores = jax.nn.sigmoid(logits)
    biased = scores + bias_ref[0:1, :]
    iota_e = lax.broadcasted_iota(_I32, (1, E), 1)
    m1 = jnp.max(biased, axis=1, keepdims=True)
    a1 = jnp.min(jnp.where(biased == m1, iota_e, E), axis=1, keepdims=True)
    masked = jnp.where(iota_e == a1, -jnp.inf, biased)
    m2 = jnp.max(masked, axis=1, keepdims=True)
    a2 = jnp.min(jnp.where(masked == m2, iota_e, E), axis=1, keepdims=True)
    s1 = jnp.sum(jnp.where(iota_e == a1, scores, 0.0), axis=1, keepdims=True)
    s2 = jnp.sum(jnp.where(iota_e == a2, scores, 0.0), axis=1, keepdims=True)
    den = s1 + s2 + 1e-20
    e1_ref[...] = a1
    e2_ref[...] = a2
    s1_ref[...] = s1 / den
    s2_ref[...] = s2 / den


def _router(x, gate_w, bias8):
    bm = 1024
    grid = (T // bm,)
    out_shape = (
        jax.ShapeDtypeStruct((T, 1), _I32),
        jax.ShapeDtypeStruct((T, 1), _I32),
        jax.ShapeDtypeStruct((T, 1), _F32),
        jax.ShapeDtypeStruct((T, 1), _F32),
    )
    row_spec = pl.BlockSpec((bm, 1), lambda i: (i, 0))
    return pl.pallas_call(
        _router_body,
        grid=grid,
        in_specs=[
            pl.BlockSpec((bm, DIM), lambda i: (i, 0)),
            pl.BlockSpec((E, DIM), lambda i: (0, 0)),
            pl.BlockSpec((8, E), lambda i: (0, 0)),
        ],
        out_specs=(row_spec, row_spec, row_spec, row_spec),
        out_shape=out_shape,
    )(x, gate_w, bias8)


def _sort_body(e1_ref, e2_ref, d1_ref, d2_ref, offs_ref, grp_ref, blk_ref,
               grpw_ref, nxt_ref):
    iota_e = lax.broadcasted_iota(_I32, (1, E), 1)
    oh1 = (e1_ref[...] == iota_e).astype(_I32)
    oh2 = (e2_ref[...] == iota_e).astype(_I32)
    c = oh1 + oh2
    s = 1
    while s < T:
        c = c + jnp.concatenate(
            [jnp.zeros((s, E), _I32), c[: T - s]], axis=0)
        s *= 2
    total = c[T - 1: T, :]
    cnt_before = c - oh1 - oh2
    oi = total
    s = 1
    while s < E:
        oi = oi + jnp.concatenate(
            [jnp.zeros((1, s), _I32), oi[:, : E - s]], axis=1)
        s *= 2
    off_excl = oi - total
    d1_ref[...] = jnp.sum(oh1 * (off_excl + cnt_before), axis=1, keepdims=True)
    d2_ref[...] = jnp.sum(oh2 * (off_excl + cnt_before), axis=1, keepdims=True)
    offs = jnp.concatenate(
        [off_excl, jnp.full((1, 2), TK, _I32)], axis=1)
    offs_ref[...] = jnp.broadcast_to(offs, (8, E + 2))
    nz = total > 0
    fb = off_excl // BM
    lb = (jnp.maximum(oi, 1) - 1) // BM
    tiles = jnp.where(nz, lb - fb + 1, 0)
    cti = tiles
    s = 1
    while s < E:
        cti = cti + jnp.concatenate(
            [jnp.zeros((1, s), _I32), cti[:, : E - s]], axis=1)
        s *= 2
    cte = cti - tiles
    item = lax.broadcasted_iota(_I32, (W, 1), 0)
    gof = jnp.sum((cti <= item).astype(_I32), axis=1, keepdims=True)
    ohg = lax.broadcasted_iota(_I32, (W, E), 1) == gof
    blk = jnp.sum(jnp.where(ohg, fb - cte, 0), axis=1, keepdims=True) + item
    blk_ref[...] = jnp.where(gof >= E, NB - 1, blk)
    grp_ref[...] = jnp.minimum(gof, E)
    iota_l = lax.broadcasted_iota(_I32, (1, E), 1)
    last_real = jnp.max(jnp.where(nz, iota_l, 0), axis=1, keepdims=True)
    grpw_ref[...] = jnp.minimum(gof, last_real)
    nxt_start = jnp.sum(jnp.where(ohg, cti, 0), axis=1, keepdims=True)
    nxt_ref[...] = jnp.sum((cti <= nxt_start).astype(_I32), axis=1,
                           keepdims=True)


def _sort(e1, e2):
    wvec = jax.ShapeDtypeStruct((W, 1), _I32)
    out_shape = (
        jax.ShapeDtypeStruct((T, 1), _I32),
        jax.ShapeDtypeStruct((T, 1), _I32),
        jax.ShapeDtypeStruct((8, E + 2), _I32),
        wvec, wvec, wvec, wvec,
    )
    return pl.pallas_call(_sort_body, out_shape=out_shape)(e1, e2)


_SC_NW = 32
_SC_CH = 16


def _sc_scatter(x, d1, d2):
    mesh = plsc.VectorSubcoreMesh(core_axis_name="c", subcore_axis_name="s")
    per_w = T // _SC_NW
    nch = per_w // _SC_CH

    @functools.partial(
        pl.kernel,
        out_type=jax.ShapeDtypeStruct((TK, DIM), _F32),
        mesh=mesh,
        scratch_types=[
            pltpu.VMEM((nch, _SC_CH), _I32),
            pltpu.VMEM((nch, _SC_CH), _I32),
            pltpu.VMEM((_SC_CH, DIM), _F32),
            pltpu.VMEM((_SC_CH, DIM), _F32),
            pltpu.SemaphoreType.DMA,
            pltpu.SemaphoreType.DMA,
            pltpu.SemaphoreType.DMA,
            pltpu.SemaphoreType.DMA,
        ],
    )
    def scatter_k(x_hbm, d1_hbm, d2_hbm, out_hbm, i1_v, i2_v, rows_a, rows_b,
                  sml_a, sml_b, sst1, sst2):
        wid = lax.axis_index("s") * 2 + lax.axis_index("c")
        base = wid * per_w
        pltpu.sync_copy(d1_hbm.at[wid], i1_v)
        pltpu.sync_copy(d2_hbm.at[wid], i2_v)
        pltpu.async_copy(x_hbm.at[pl.ds(base, _SC_CH)], rows_a, sml_a)

        def round_(j, cur, oth, sml_cur, sml_oth):
            b = base + j * _SC_CH
            pltpu.make_async_copy(
                x_hbm.at[pl.ds(b, _SC_CH)], cur, sml_cur).wait()

            @pl.when(j + 1 < nch)
            def _():
                @pl.when(j >= 1)
                def _():
                    pltpu.make_async_copy(
                        oth, out_hbm.at[pl.ds(0, _SC_CH)], sst1).wait()
                    pltpu.make_async_copy(
                        oth, out_hbm.at[pl.ds(0, _SC_CH)], sst2).wait()

                pltpu.async_copy(
                    x_hbm.at[pl.ds(b + _SC_CH, _SC_CH)], oth, sml_oth)

            pltpu.async_copy(cur, out_hbm.at[i1_v.at[j]], sst1)
            pltpu.async_copy(cur, out_hbm.at[i2_v.at[j]], sst2)

        def body(j, carry):
            @pl.when(j % 2 == 0)
            def _():
                round_(j, rows_a, rows_b, sml_a, sml_b)

            @pl.when(j % 2 == 1)
            def _():
                round_(j, rows_b, rows_a, sml_b, sml_a)
            return carry

        lax.fori_loop(0, nch, body, 0)

        def drain(j, carry):
            pltpu.make_async_copy(rows_a, out_hbm.at[pl.ds(0, _SC_CH)],
                                  sst1).wait()
            pltpu.make_async_copy(rows_a, out_hbm.at[pl.ds(0, _SC_CH)],
                                  sst2).wait()
            return carry

        lax.fori_loop(0, 2, drain, 0)

    return scatter_k(x, d1, d2)


def _sc_gather(eo, d1, d2):
    mesh = plsc.VectorSubcoreMesh(core_axis_name="c", subcore_axis_name="s")
    per_w = T // _SC_NW
    nch = per_w // _SC_CH
    row_t = jax.ShapeDtypeStruct((T, DIM), _F32)

    @functools.partial(
        pl.kernel,
        out_type=(row_t, row_t),
        mesh=mesh,
        scratch_types=[
            pltpu.VMEM((nch, _SC_CH), _I32),
            pltpu.VMEM((nch, _SC_CH), _I32),
            pltpu.VMEM((_SC_CH, DIM), _F32),
            pltpu.VMEM((_SC_CH, DIM), _F32),
            pltpu.SemaphoreType.DMA,
            pltpu.SemaphoreType.DMA,
            pltpu.SemaphoreType.DMA,
            pltpu.SemaphoreType.DMA,
        ],
    )
    def gather_k(eo_hbm, d1_hbm, d2_hbm, g1_hbm, g2_hbm, i1_v, i2_v,
                 rows_a, rows_b, sg1, sg2, sst1, sst2):
        wid = lax.axis_index("s") * 2 + lax.axis_index("c")
        base = wid * per_w
        pltpu.sync_copy(d1_hbm.at[wid], i1_v)
        pltpu.sync_copy(d2_hbm.at[wid], i2_v)

        def body(j, carry):
            b = base + j * _SC_CH
            pltpu.async_copy(eo_hbm.at[i1_v.at[j]], rows_a, sg1)
            pltpu.async_copy(eo_hbm.at[i2_v.at[j]], rows_b, sg2)
            pltpu.make_async_copy(eo_hbm.at[pl.ds(0, _SC_CH)], rows_a,
                                  sg1).wait()
            pltpu.async_copy(rows_a, g1_hbm.at[pl.ds(b, _SC_CH)], sst1)
            pltpu.make_async_copy(eo_hbm.at[pl.ds(0, _SC_CH)], rows_b,
                                  sg2).wait()
            pltpu.async_copy(rows_b, g2_hbm.at[pl.ds(b, _SC_CH)], sst2)
            pltpu.make_async_copy(rows_a, g1_hbm.at[pl.ds(b, _SC_CH)],
                                  sst1).wait()
            pltpu.make_async_copy(rows_b, g2_hbm.at[pl.ds(b, _SC_CH)],
                                  sst2).wait()
            return carry

        lax.fori_loop(0, nch, body, 0)

    return gather_k(eo, d1, d2)


def _ffn_body(offs_ref, grp_ref, blk_ref, grpw_ref, nxt_ref,
              xs_ref, w1_ref, w3_ref, w2_ref, out_ref,
              w1f_ref, w3f_ref, w2f_ref, w1s_ref, w3s_ref, w2s_ref,
              sem1, sem3, sem2):
    w = pl.program_id(0)
    g = grp_ref[w]
    st = offs_ref[g]
    en = offs_ref[g + 1]
    b = blk_ref[w]
    wprev = jnp.maximum(w - 1, 0)
    gw = grpw_ref[w]

    @pl.when(w == 0)
    def _():
        pltpu.async_copy(w1_ref.at[gw], w1f_ref, sem1)
        pltpu.async_copy(w3_ref.at[gw], w3f_ref, sem3)
        pltpu.async_copy(w2_ref.at[gw], w2f_ref, sem2)

    @pl.when((w == 0) | (gw != grpw_ref[wprev]))
    def _():
        pltpu.make_async_copy(w1_ref.at[gw], w1f_ref, sem1).wait()
        pltpu.make_async_copy(w3_ref.at[gw], w3f_ref, sem3).wait()
        pltpu.make_async_copy(w2_ref.at[gw], w2f_ref, sem2).wait()
        w1s_ref[...] = w1f_ref[...].astype(_BF16)
        w3s_ref[...] = w3f_ref[...].astype(_BF16)
        w2s_ref[...] = w2f_ref[...].astype(_BF16)
        nx = nxt_ref[w]

        @pl.when(nx < E)
        def _():
            pltpu.async_copy(w1_ref.at[nx], w1f_ref, sem1)
            pltpu.async_copy(w3_ref.at[nx], w3f_ref, sem3)
            pltpu.async_copy(w2_ref.at[nx], w2f_ref, sem2)

    @pl.when(st < en)
    def _():
        rid = b * BM + lax.broadcasted_iota(_I32, (BM, 1), 0)
        mask = (rid >= st) & (rid < en)
        x = xs_ref[...].astype(_BF16)
        a = _mm_t(x, w1s_ref[...])
        c3 = _mm_t(x, w3s_ref[...])
        h = ((a * jax.nn.sigmoid(a)) * c3).astype(_BF16)
        oe = _mm_t(h, w2s_ref[...])
        contrib = jnp.where(mask, oe, 0.0)
        first = (w == 0) | (b != blk_ref[wprev])

        @pl.when(first)
        def _():
            out_ref[...] = contrib

        @pl.when(jnp.logical_not(first))
        def _():
            out_ref[...] += contrib


def _grouped_ffn(xs, w1, w3, w2, offs, grp, blk, grpw, nxt):
    grid_spec = pltpu.PrefetchScalarGridSpec(
        num_scalar_prefetch=5,
        grid=(W,),
        in_specs=[
            pl.BlockSpec((BM, DIM), lambda w, o, g, b, gw, nx: (b[w], 0)),
            pl.BlockSpec(memory_space=pl.ANY),
            pl.BlockSpec(memory_space=pl.ANY),
            pl.BlockSpec(memory_space=pl.ANY),
        ],
        out_specs=pl.BlockSpec((BM, DIM), lambda w, o, g, b, gw, nx: (b[w], 0)),
        scratch_shapes=[
            pltpu.VMEM((HID, DIM), _F32),
            pltpu.VMEM((HID, DIM), _F32),
            pltpu.VMEM((DIM, HID), _F32),
            pltpu.VMEM((HID, DIM), _BF16),
            pltpu.VMEM((HID, DIM), _BF16),
            pltpu.VMEM((DIM, HID), _BF16),
            pltpu.SemaphoreType.DMA,
            pltpu.SemaphoreType.DMA,
            pltpu.SemaphoreType.DMA,
        ],
    )
    return pl.pallas_call(
        _ffn_body,
        grid_spec=grid_spec,
        out_shape=jax.ShapeDtypeStruct((TK, DIM), _F32),
        compiler_params=pltpu.CompilerParams(
            dimension_semantics=("arbitrary",),
            vmem_limit_bytes=120 * 1024 * 1024,
        ),
    )(offs, grp, blk, grpw, nxt, xs, w1, w3, w2)


def _shared_body(x_ref, ws1_ref, ws3_ref, ws2_ref, out_ref,
                 ws1s_ref, ws3s_ref, ws2s_ref):
    @pl.when(pl.program_id(0) == 0)
    def _():
        ws1s_ref[...] = ws1_ref[...].astype(_BF16)
        ws3s_ref[...] = ws3_ref[...].astype(_BF16)
        ws2s_ref[...] = ws2_ref[...].astype(_BF16)

    x = x_ref[...].astype(_BF16)
    a = _mm_t(x, ws1s_ref[...])
    c3 = _mm_t(x, ws3s_ref[...])
    h = ((a * jax.nn.sigmoid(a)) * c3).astype(_BF16)
    out_ref[...] = _mm_t(h, ws2s_ref[...])


def _shared_ffn(x, ws1, ws3, ws2):
    bm = 512
    grid = (T // bm,)
    row_spec = pl.BlockSpec((bm, DIM), lambda i: (i, 0))
    return pl.pallas_call(
        _shared_body,
        grid=grid,
        in_specs=[
            row_spec,
            pl.BlockSpec((HID, DIM), lambda i: (0, 0)),
            pl.BlockSpec((HID, DIM), lambda i: (0, 0)),
            pl.BlockSpec((DIM, HID), lambda i: (0, 0)),
        ],
        out_specs=row_spec,
        out_shape=jax.ShapeDtypeStruct((T, DIM), _F32),
        scratch_shapes=[
            pltpu.VMEM((HID, DIM), _BF16),
            pltpu.VMEM((HID, DIM), _BF16),
            pltpu.VMEM((DIM, HID), _BF16),
        ],
        compiler_params=pltpu.CompilerParams(
            vmem_limit_bytes=120 * 1024 * 1024,
        ),
    )(x, ws1, ws3, ws2)


def _combine_body(sh_ref, g1_ref, g2_ref, s1_ref, s2_ref, out_ref):
    out_ref[...] = (sh_ref[...] + s1_ref[...] * g1_ref[...]
                    + s2_ref[...] * g2_ref[...])


def _combine(sh, g1, g2, s1, s2):
    bm = 512
    grid = (T // bm,)
    row_spec = pl.BlockSpec((bm, DIM), lambda i: (i, 0))
    s_spec = pl.BlockSpec((bm, 1), lambda i: (i, 0))
    return pl.pallas_call(
        _combine_body,
        grid=grid,
        in_specs=[row_spec, row_spec, row_spec, s_spec, s_spec],
        out_specs=row_spec,
        out_shape=jax.ShapeDtypeStruct((T, DIM), _F32),
    )(sh, g1, g2, s1, s2)


def kernel(x, gate_w, w1, w2, w3, ws1, ws2, ws3, expert_bias):
    bias8 = jnp.broadcast_to(expert_bias[None, :], (8, E))
    e1, e2, s1, s2 = _router(x, gate_w, bias8)
    d1, d2, offs8, grp, blk, grpw, nxt = _sort(e1, e2)
    nch = (T // _SC_NW) // _SC_CH
    d1f = d1.reshape(_SC_NW, nch, _SC_CH)
    d2f = d2.reshape(_SC_NW, nch, _SC_CH)
    offs = offs8[0]
    xs = _sc_scatter(x, d1f, d2f)
    eo = _grouped_ffn(xs, w1, w3, w2, offs,
                      grp.reshape(W), blk.reshape(W),
                      grpw.reshape(W), nxt.reshape(W))
    g1, g2 = _sc_gather(eo, d1f, d2f)
    sh = _shared_ffn(x, ws1, ws3, ws2)
    return _combine(sh, g1, g2, s1, s2)
